# Initial kernel scaffold; baseline (speedup 1.0000x reference)
import functools

import jax
import jax.numpy as jnp
from jax import lax
from jax.experimental import pallas as pl
from jax.experimental.pallas import tpu as pltpu
from jax.experimental.pallas import tpu_sc as plsc

N = 10000
E = 320000
D = 128

NC = 2
NS = 16
NW = NC * NS
EPW = E // NW
CH = 80
NCHUNK = EPW // CH
RPT = N // NS
ZR = 125

_MESH = plsc.VectorSubcoreMesh(
    core_axis_name="c", subcore_axis_name="s", num_cores=NC, num_subcores=NS
)


def _agg_body(with_cnt, *refs):
    if with_cnt:
        (x_hbm, src_hbm, dst_hbm, z_hbm, z16_hbm, ones_hbm,
         agg_hbm, cnt_hbm,
         idx_s, idx_d, rows, zbuf, czbuf, ones_v, acc, cnt_acc) = refs
    else:
        (x_hbm, src_hbm, dst_hbm, z_hbm,
         agg_hbm,
         idx_s, idx_d, rows, zbuf, acc) = refs

    c = lax.axis_index("c")
    s = lax.axis_index("s")
    wid = c * NS + s
    row0 = s * RPT

    pltpu.sync_copy(z_hbm, zbuf)

    def zinit(j, _):
        pltpu.sync_copy(zbuf, acc.at[pl.ds(row0 + j * ZR, ZR)])
        return ()

    lax.fori_loop(0, RPT // ZR, zinit, ())
    if with_cnt:
        pltpu.sync_copy(z16_hbm, czbuf)
        pltpu.sync_copy(czbuf, cnt_acc.at[pl.ds(row0, RPT)])
        pltpu.sync_copy(ones_hbm, ones_v)
    plsc.subcore_barrier()

    def chunk(i, _):
        base = wid * EPW + i * CH
        pltpu.sync_copy(src_hbm.at[pl.ds(base, CH)], idx_s)
        pltpu.sync_copy(dst_hbm.at[pl.ds(base, CH)], idx_d)
        pltpu.sync_copy(x_hbm.at[idx_s], rows)
        pltpu.sync_copy(rows, acc.at[idx_d], add=True)
        if with_cnt:
            pltpu.sync_copy(ones_v, cnt_acc.at[idx_d], add=True)
        return ()

    lax.fori_loop(0, NCHUNK, chunk, ())
    plsc.subcore_barrier()

    def drain(j, _):
        r = row0 + j * ZR
        pltpu.sync_copy(acc.at[pl.ds(r, ZR)], zbuf)
        pltpu.sync_copy(zbuf, agg_hbm.at[c, pl.ds(r, ZR)])
        return ()

    lax.fori_loop(0, RPT // ZR, drain, ())
    if with_cnt:
        pltpu.sync_copy(cnt_acc.at[pl.ds(row0, RPT)], czbuf)
        pltpu.sync_copy(czbuf, cnt_hbm.at[c, pl.ds(row0, RPT)])


_agg_cnt_call = pl.kernel(
    functools.partial(_agg_body, True),
    out_type=(
        jax.ShapeDtypeStruct((NC, N, D), jnp.float32),
        jax.ShapeDtypeStruct((NC, N, 16), jnp.float32),
    ),
    mesh=_MESH,
    scratch_types=[
        pltpu.VMEM((CH,), jnp.int32),
        pltpu.VMEM((CH,), jnp.int32),
        pltpu.VMEM((CH, D), jnp.float32),
        pltpu.VMEM((ZR, D), jnp.float32),
        pltpu.VMEM((RPT, 16), jnp.float32),
        pltpu.VMEM((CH, 16), jnp.float32),
        pltpu.VMEM_SHARED((N, D), jnp.float32),
        pltpu.VMEM_SHARED((N, 16), jnp.float32),
    ],
)

_agg_call = pl.kernel(
    functools.partial(_agg_body, False),
    out_type=jax.ShapeDtypeStruct((NC, N, D), jnp.float32),
    mesh=_MESH,
    scratch_types=[
        pltpu.VMEM((CH,), jnp.int32),
        pltpu.VMEM((CH,), jnp.int32),
        pltpu.VMEM((CH, D), jnp.float32),
        pltpu.VMEM((ZR, D), jnp.float32),
        pltpu.VMEM_SHARED((N, D), jnp.float32),
    ],
)


def _fuse_body(relu, aggp_ref, cntp_ref, x_ref, wlt_ref, bl_ref, wrt_ref, o_ref):
    cnt = cntp_ref[0, :, 0:1] + cntp_ref[1, :, 0:1]
    mean = (aggp_ref[0] + aggp_ref[1]) / jnp.maximum(cnt, 1.0)
    y = jnp.dot(mean, wlt_ref[...], preferred_element_type=jnp.float32)
    y = y + jnp.dot(x_ref[...], wrt_ref[...], preferred_element_type=jnp.float32)
    y = y + bl_ref[...]
    o_ref[...] = jnp.maximum(y, 0.0) if relu else y


def _fuse(relu, aggp, cntp, x, wlt, bl2, wrt):
    BN = 1000
    return pl.pallas_call(
        functools.partial(_fuse_body, relu),
        grid=(N // BN,),
        in_specs=[
            pl.BlockSpec((NC, BN, D), lambda i: (0, i, 0)),
            pl.BlockSpec((NC, BN, 16), lambda i: (0, i, 0)),
            pl.BlockSpec((BN, D), lambda i: (i, 0)),
            pl.BlockSpec((D, D), lambda i: (0, 0)),
            pl.BlockSpec((1, D), lambda i: (0, 0)),
            pl.BlockSpec((D, D), lambda i: (0, 0)),
        ],
        out_specs=pl.BlockSpec((BN, D), lambda i: (i, 0)),
        out_shape=jax.ShapeDtypeStruct((N, D), jnp.float32),
    )(aggp, cntp, x, wlt, bl2, wrt)


def kernel(x, edge_index, Wl1, bl1, Wr1, Wl2, bl2, Wr2, Wl3, bl3, Wr3):
    src = edge_index[0].astype(jnp.int32)
    dst = edge_index[1].astype(jnp.int32)
    zeros = jnp.zeros((ZR, D), jnp.float32)
    zeros16 = jnp.zeros((RPT, 16), jnp.float32)
    ones = jnp.ones((CH, 16), jnp.float32)

    aggp1, cntp = _agg_cnt_call(x, src, dst, zeros, zeros16, ones)
    h1 = _fuse(True, aggp1, cntp, x, Wl1.T, bl1[None, :], Wr1.T)
    aggp2 = _agg_call(h1, src, dst, zeros)
    h2 = _fuse(True, aggp2, cntp, h1, Wl2.T, bl2[None, :], Wr2.T)
    aggp3 = _agg_call(h2, src, dst, zeros)
    return _fuse(False, aggp3, cntp, h2, Wl3.T, bl3[None, :], Wr3.T)


# speedup vs baseline: 4.7686x; 4.7686x over previous
import functools

import jax
import jax.numpy as jnp
from jax import lax
from jax.experimental import pallas as pl
from jax.experimental.pallas import tpu as pltpu
from jax.experimental.pallas import tpu_sc as plsc

N = 10000
E = 320000
D = 128

NC = 2
NS = 16
NW = NC * NS
EPW = E // NW
CH = 80
NCHUNK = EPW // CH
NP = 10240
RPT = NP // NS
ZR = 128

_MESH = plsc.VectorSubcoreMesh(
    core_axis_name="c", subcore_axis_name="s", num_cores=NC, num_subcores=NS
)


def _agg_body(x_hbm, src_hbm, dst_hbm, z_hbm, agg_hbm,
              idx_s, idx_d, rows, zbuf, acc):
    c = lax.axis_index("c")
    s = lax.axis_index("s")
    wid = c * NS + s
    row0 = s * RPT

    pltpu.sync_copy(z_hbm, zbuf)

    def zinit(j, _):
        pltpu.sync_copy(zbuf, acc.at[pl.ds(row0 + j * ZR, ZR)])
        return ()

    lax.fori_loop(0, RPT // ZR, zinit, ())
    plsc.subcore_barrier()

    def chunk(i, _):
        base = wid * EPW + i * CH
        pltpu.sync_copy(src_hbm.at[pl.ds(base, CH)], idx_s)
        pltpu.sync_copy(dst_hbm.at[pl.ds(base, CH)], idx_d)
        pltpu.sync_copy(x_hbm.at[idx_s], rows)
        pltpu.sync_copy(rows, acc.at[idx_d], add=True)
        return ()

    lax.fori_loop(0, NCHUNK, chunk, ())
    plsc.subcore_barrier()

    def drain(j, _):
        r = row0 + j * ZR
        pltpu.sync_copy(acc.at[pl.ds(r, ZR)], zbuf)
        pltpu.sync_copy(zbuf, agg_hbm.at[c, pl.ds(r, ZR)])
        return ()

    lax.fori_loop(0, RPT // ZR, drain, ())


_agg_call = pl.kernel(
    _agg_body,
    out_type=jax.ShapeDtypeStruct((NC, NP, D), jnp.float32),
    mesh=_MESH,
    scratch_types=[
        pltpu.VMEM((CH,), jnp.int32),
        pltpu.VMEM((CH,), jnp.int32),
        pltpu.VMEM((CH, D), jnp.float32),
        pltpu.VMEM((ZR, D), jnp.float32),
        pltpu.VMEM_SHARED((NP, D), jnp.float32),
    ],
)


def _cnt_body(dst_hbm, z_hbm, ones_hbm, cnt_hbm,
              idx_d, zbuf, ones_v, cnt_acc):
    c = lax.axis_index("c")
    s = lax.axis_index("s")
    wid = c * NS + s
    row0 = s * RPT

    pltpu.sync_copy(z_hbm, zbuf)

    def zinit(j, _):
        pltpu.sync_copy(zbuf, cnt_acc.at[pl.ds(row0 + j * ZR, ZR)])
        return ()

    lax.fori_loop(0, RPT // ZR, zinit, ())
    pltpu.sync_copy(ones_hbm, ones_v)
    plsc.subcore_barrier()

    def chunk(i, _):
        base = wid * EPW + i * CH
        pltpu.sync_copy(dst_hbm.at[pl.ds(base, CH)], idx_d)
        pltpu.sync_copy(ones_v, cnt_acc.at[idx_d], add=True)
        return ()

    lax.fori_loop(0, NCHUNK, chunk, ())
    plsc.subcore_barrier()

    def drain(j, _):
        r = row0 + j * ZR
        pltpu.sync_copy(cnt_acc.at[pl.ds(r, ZR)], zbuf)
        pltpu.sync_copy(zbuf, cnt_hbm.at[c, pl.ds(r, ZR)])
        return ()

    lax.fori_loop(0, RPT // ZR, drain, ())


_cnt_call = pl.kernel(
    _cnt_body,
    out_type=jax.ShapeDtypeStruct((NC, NP, D), jnp.float32),
    mesh=_MESH,
    scratch_types=[
        pltpu.VMEM((CH,), jnp.int32),
        pltpu.VMEM((ZR, D), jnp.float32),
        pltpu.VMEM((CH, D), jnp.float32),
        pltpu.VMEM_SHARED((NP, D), jnp.float32),
    ],
)


def _fuse_body(relu, aggp_ref, cntp_ref, x_ref, wlt_ref, bl_ref, wrt_ref, o_ref):
    cnt = cntp_ref[0, :, 0:1] + cntp_ref[1, :, 0:1]
    mean = (aggp_ref[0] + aggp_ref[1]) / jnp.maximum(cnt, 1.0)
    y = jnp.dot(mean, wlt_ref[...], preferred_element_type=jnp.float32)
    y = y + jnp.dot(x_ref[...], wrt_ref[...], preferred_element_type=jnp.float32)
    y = y + bl_ref[...]
    o_ref[...] = jnp.maximum(y, 0.0) if relu else y


def _fuse(relu, aggp, cntp, x, wlt, bl2, wrt):
    BN = 1000
    return pl.pallas_call(
        functools.partial(_fuse_body, relu),
        grid=(N // BN,),
        in_specs=[
            pl.BlockSpec((NC, BN, D), lambda i: (0, i, 0)),
            pl.BlockSpec((NC, BN, D), lambda i: (0, i, 0)),
            pl.BlockSpec((BN, D), lambda i: (i, 0)),
            pl.BlockSpec((D, D), lambda i: (0, 0)),
            pl.BlockSpec((1, D), lambda i: (0, 0)),
            pl.BlockSpec((D, D), lambda i: (0, 0)),
        ],
        out_specs=pl.BlockSpec((BN, D), lambda i: (i, 0)),
        out_shape=jax.ShapeDtypeStruct((N, D), jnp.float32),
    )(aggp, cntp, x, wlt, bl2, wrt)


def kernel(x, edge_index, Wl1, bl1, Wr1, Wl2, bl2, Wr2, Wl3, bl3, Wr3):
    src = edge_index[0].astype(jnp.int32)
    dst = edge_index[1].astype(jnp.int32)
    zeros = jnp.zeros((ZR, D), jnp.float32)
    ones = jnp.ones((CH, D), jnp.float32)

    cntp = _cnt_call(dst, zeros, ones)
    aggp1 = _agg_call(x, src, dst, zeros)
    h1 = _fuse(True, aggp1, cntp, x, Wl1.T, bl1[None, :], Wr1.T)
    aggp2 = _agg_call(h1, src, dst, zeros)
    h2 = _fuse(True, aggp2, cntp, h1, Wl2.T, bl2[None, :], Wr2.T)
    aggp3 = _agg_call(h2, src, dst, zeros)
    return _fuse(False, aggp3, cntp, h2, Wl3.T, bl3[None, :], Wr3.T)
